# baseline (speedup 1.0000x reference)
import functools

import jax
import jax.numpy as jnp
from jax import lax
from jax.experimental import pallas as pl
from jax.experimental.pallas import tpu as pltpu
from jax.experimental.pallas import tpu_sc as plsc

B, M, V = 32, 16384, 64
NC, NS, L = 2, 16, 16
C = 512
CW = C * V
NCH = M // C


def _make_kernel():
    mesh = plsc.VectorSubcoreMesh(core_axis_name="c", subcore_axis_name="s")

    @functools.partial(
        pl.kernel,
        mesh=mesh,
        out_type=jax.ShapeDtypeStruct((B, V), jnp.float32),
        scratch_types=[
            pltpu.VMEM((CW,), jnp.float32),
            pltpu.VMEM((CW,), jnp.float32),
            pltpu.VMEM((M,), jnp.float32),
            pltpu.VMEM((V,), jnp.float32),
            pltpu.VMEM((V,), jnp.float32),
            pltpu.SemaphoreType.DMA,
            pltpu.SemaphoreType.DMA,
        ],
    )
    def vm_kernel(w_hbm, v_hbm, mem_hbm, out_hbm,
                  buf0, buf1, w_v, v_v, o_v, sem0, sem1):
        b = lax.axis_index("s") * NC + lax.axis_index("c")

        pltpu.sync_copy(w_hbm.at[b], w_v)
        pltpu.sync_copy(v_hbm.at[b], v_v)

        def wsq_body(i, acc):
            x = w_v[pl.ds(i * L, L)]
            return acc + x * x

        wsqv = lax.fori_loop(0, M // L, wsq_body, jnp.zeros((L,), jnp.float32))
        wsq = wsqv[0]
        for i in range(1, L):
            wsq = wsq + wsqv[i]

        def rows(buf, base, accs):
            def row_body(m16, accs):
                a0, a1, a2, a3 = accs
                wv = w_v[pl.ds(base + m16 * L, L)]
                for i in range(L):
                    ws = wv[i]
                    r = (m16 * L + i) * V
                    a0 = a0 + ws * buf[pl.ds(r + 0 * L, L)]
                    a1 = a1 + ws * buf[pl.ds(r + 1 * L, L)]
                    a2 = a2 + ws * buf[pl.ds(r + 2 * L, L)]
                    a3 = a3 + ws * buf[pl.ds(r + 3 * L, L)]
                return (a0, a1, a2, a3)

            return lax.fori_loop(0, C // L, row_body, accs)

        pltpu.async_copy(mem_hbm.at[b, pl.ds(0, CW)], buf0, sem0)

        def chunk_body(g, accs):
            c0 = g * 2
            pltpu.async_copy(mem_hbm.at[b, pl.ds((c0 + 1) * CW, CW)], buf1, sem1)
            pltpu.make_async_copy(mem_hbm.at[b, pl.ds(0, CW)], buf0, sem0).wait()
            accs = rows(buf0, c0 * C, accs)

            @pl.when(c0 + 2 < NCH)
            def _():
                pltpu.async_copy(mem_hbm.at[b, pl.ds((c0 + 2) * CW, CW)], buf0, sem0)

            pltpu.make_async_copy(mem_hbm.at[b, pl.ds(0, CW)], buf1, sem1).wait()
            accs = rows(buf1, (c0 + 1) * C, accs)
            return accs

        z = jnp.zeros((L,), jnp.float32)
        a0, a1, a2, a3 = lax.fori_loop(0, NCH // 2, chunk_body, (z, z, z, z))

        o_v[pl.ds(0 * L, L)] = a0 + wsq * v_v[pl.ds(0 * L, L)]
        o_v[pl.ds(1 * L, L)] = a1 + wsq * v_v[pl.ds(1 * L, L)]
        o_v[pl.ds(2 * L, L)] = a2 + wsq * v_v[pl.ds(2 * L, L)]
        o_v[pl.ds(3 * L, L)] = a3 + wsq * v_v[pl.ds(3 * L, L)]
        pltpu.sync_copy(o_v, out_hbm.at[b])

    return vm_kernel


_vm_kernel = _make_kernel()


def kernel(w, v, memory):
    mem_flat = memory.reshape(B, M * V)
    return _vm_kernel(w, v, mem_flat)


# speedup vs baseline: 1.0336x; 1.0336x over previous
import functools

import jax
import jax.numpy as jnp
from jax import lax
from jax.experimental import pallas as pl
from jax.experimental.pallas import tpu as pltpu
from jax.experimental.pallas import tpu_sc as plsc

B, M, V = 32, 16384, 64
NC, NS, L = 2, 16, 16
C = 512
CW = C * V
NCH = M // C


def _make_kernel():
    mesh = plsc.VectorSubcoreMesh(core_axis_name="c", subcore_axis_name="s")

    @functools.partial(
        pl.kernel,
        mesh=mesh,
        out_type=jax.ShapeDtypeStruct((B, V), jnp.float32),
        scratch_types=[
            pltpu.VMEM((CW,), jnp.float32),
            pltpu.VMEM((CW,), jnp.float32),
            pltpu.VMEM((M,), jnp.float32),
            pltpu.VMEM((V,), jnp.float32),
            pltpu.VMEM((V,), jnp.float32),
            pltpu.SemaphoreType.DMA,
            pltpu.SemaphoreType.DMA,
        ],
    )
    def vm_kernel(w_hbm, v_hbm, mem_hbm, out_hbm,
                  buf0, buf1, w_v, v_v, o_v, sem0, sem1):
        b = lax.axis_index("s") * NC + lax.axis_index("c")

        pltpu.sync_copy(w_hbm.at[b], w_v)
        pltpu.sync_copy(v_hbm.at[b], v_v)

        def wsq_body(i, acc):
            x = w_v[pl.ds(i * L, L)]
            return acc + x * x

        wsqv = lax.fori_loop(0, M // L, wsq_body, jnp.zeros((L,), jnp.float32))
        wsq = wsqv[0]
        for i in range(1, L):
            wsq = wsq + wsqv[i]

        _dnums = lax.GatherDimensionNumbers(
            offset_dims=(), collapsed_slice_dims=(0,), start_index_map=(0,))

        def splat(x, i):
            idx = jnp.full((L, 1), i, jnp.int32)
            return lax.gather(x, idx, _dnums, slice_sizes=(1,),
                              mode=lax.GatherScatterMode.PROMISE_IN_BOUNDS)

        def rows(buf, base, accs):
            def row_body(m16, accs):
                a0, a1, a2, a3 = accs
                wv = w_v[pl.ds(base + m16 * L, L)]
                for i in range(L):
                    wb = splat(wv, i)
                    r = (m16 * L + i) * V
                    a0 = a0 + wb * buf[pl.ds(r + 0 * L, L)]
                    a1 = a1 + wb * buf[pl.ds(r + 1 * L, L)]
                    a2 = a2 + wb * buf[pl.ds(r + 2 * L, L)]
                    a3 = a3 + wb * buf[pl.ds(r + 3 * L, L)]
                return (a0, a1, a2, a3)

            return lax.fori_loop(0, C // L, row_body, accs)

        pltpu.async_copy(mem_hbm.at[b, pl.ds(0, CW)], buf0, sem0)

        def chunk_body(g, accs):
            c0 = g * 2
            pltpu.async_copy(mem_hbm.at[b, pl.ds((c0 + 1) * CW, CW)], buf1, sem1)
            pltpu.make_async_copy(mem_hbm.at[b, pl.ds(0, CW)], buf0, sem0).wait()
            accs = rows(buf0, c0 * C, accs)

            @pl.when(c0 + 2 < NCH)
            def _():
                pltpu.async_copy(mem_hbm.at[b, pl.ds((c0 + 2) * CW, CW)], buf0, sem0)

            pltpu.make_async_copy(mem_hbm.at[b, pl.ds(0, CW)], buf1, sem1).wait()
            accs = rows(buf1, (c0 + 1) * C, accs)
            return accs

        z = jnp.zeros((L,), jnp.float32)
        a0, a1, a2, a3 = lax.fori_loop(0, NCH // 2, chunk_body, (z, z, z, z))

        o_v[pl.ds(0 * L, L)] = a0 + wsq * v_v[pl.ds(0 * L, L)]
        o_v[pl.ds(1 * L, L)] = a1 + wsq * v_v[pl.ds(1 * L, L)]
        o_v[pl.ds(2 * L, L)] = a2 + wsq * v_v[pl.ds(2 * L, L)]
        o_v[pl.ds(3 * L, L)] = a3 + wsq * v_v[pl.ds(3 * L, L)]
        pltpu.sync_copy(o_v, out_hbm.at[b])

    return vm_kernel


_vm_kernel = _make_kernel()


def kernel(w, v, memory):
    mem_flat = memory.reshape(B, M * V)
    return _vm_kernel(w, v, mem_flat)


# speedup vs baseline: 1.2044x; 1.1653x over previous
import functools

import jax
import jax.numpy as jnp
from jax import lax
from jax.experimental import pallas as pl
from jax.experimental.pallas import tpu as pltpu
from jax.experimental.pallas import tpu_sc as plsc

B, M, V = 32, 16384, 64
NC, NS, L = 2, 16, 16
C = 256
NCH = M // C


def _make_kernel():
    mesh = plsc.VectorSubcoreMesh(core_axis_name="c", subcore_axis_name="s")

    @functools.partial(
        pl.kernel,
        mesh=mesh,
        out_type=jax.ShapeDtypeStruct((B, V), jnp.float32),
        scratch_types=[
            pltpu.VMEM((C, V), jnp.float32),
            pltpu.VMEM((C, V), jnp.float32),
            pltpu.VMEM((M,), jnp.float32),
            pltpu.VMEM((V,), jnp.float32),
            pltpu.VMEM((V,), jnp.float32),
            pltpu.SemaphoreType.DMA,
            pltpu.SemaphoreType.DMA,
        ],
    )
    def vm_kernel(w_hbm, v_hbm, mem_hbm, out_hbm,
                  buf0, buf1, w_v, v_v, o_v, sem0, sem1):
        b = lax.axis_index("s") * NC + lax.axis_index("c")

        pltpu.sync_copy(w_hbm.at[b], w_v)
        pltpu.sync_copy(v_hbm.at[b], v_v)

        def wsq_body(i, acc):
            x = w_v[pl.ds(i * L, L)]
            return acc + x * x

        wsqv = lax.fori_loop(0, M // L, wsq_body, jnp.zeros((L,), jnp.float32))
        wsq = wsqv[0]
        for i in range(1, L):
            wsq = wsq + wsqv[i]

        _dnums = lax.GatherDimensionNumbers(
            offset_dims=(), collapsed_slice_dims=(0,), start_index_map=(0,))

        def splat(x, i):
            idx = jnp.full((L, 1), i, jnp.int32)
            return lax.gather(x, idx, _dnums, slice_sizes=(1,),
                              mode=lax.GatherScatterMode.PROMISE_IN_BOUNDS)

        def rows(buf, base, accs):
            def row_body(m16, accs):
                a0, a1, a2, a3 = accs
                wv = w_v[pl.ds(base + m16 * L, L)]
                for i in range(L):
                    wb = splat(wv, i)
                    r = m16 * L + i
                    a0 = a0 + wb * buf[r, pl.ds(0 * L, L)]
                    a1 = a1 + wb * buf[r, pl.ds(1 * L, L)]
                    a2 = a2 + wb * buf[r, pl.ds(2 * L, L)]
                    a3 = a3 + wb * buf[r, pl.ds(3 * L, L)]
                return (a0, a1, a2, a3)

            return lax.fori_loop(0, C // L, row_body, accs)

        pltpu.async_copy(mem_hbm.at[b, pl.ds(0, C), :], buf0, sem0)

        def chunk_body(g, accs):
            c0 = g * 2
            pltpu.async_copy(mem_hbm.at[b, pl.ds((c0 + 1) * C, C), :], buf1, sem1)
            pltpu.make_async_copy(mem_hbm.at[b, pl.ds(0, C), :], buf0, sem0).wait()
            accs = rows(buf0, c0 * C, accs)

            @pl.when(c0 + 2 < NCH)
            def _():
                pltpu.async_copy(mem_hbm.at[b, pl.ds((c0 + 2) * C, C), :], buf0, sem0)

            pltpu.make_async_copy(mem_hbm.at[b, pl.ds(0, C), :], buf1, sem1).wait()
            accs = rows(buf1, (c0 + 1) * C, accs)
            return accs

        z = jnp.zeros((L,), jnp.float32)
        a0, a1, a2, a3 = lax.fori_loop(0, NCH // 2, chunk_body, (z, z, z, z))

        o_v[pl.ds(0 * L, L)] = a0 + wsq * v_v[pl.ds(0 * L, L)]
        o_v[pl.ds(1 * L, L)] = a1 + wsq * v_v[pl.ds(1 * L, L)]
        o_v[pl.ds(2 * L, L)] = a2 + wsq * v_v[pl.ds(2 * L, L)]
        o_v[pl.ds(3 * L, L)] = a3 + wsq * v_v[pl.ds(3 * L, L)]
        pltpu.sync_copy(o_v, out_hbm.at[b])

    return vm_kernel


_vm_kernel = _make_kernel()


def kernel(w, v, memory):
    return _vm_kernel(w, v, memory)


# speedup vs baseline: 4.5671x; 3.7921x over previous
import functools

import jax
import jax.numpy as jnp
from jax import lax
from jax.experimental import pallas as pl
from jax.experimental.pallas import tpu as pltpu
from jax.experimental.pallas import tpu_sc as plsc

B, M, V = 32, 16384, 64
NC, NS, L = 2, 16, 16
MC = 4096
NCH = (V // 8) * (M // MC)


def _make_kernel():
    mesh = plsc.VectorSubcoreMesh(core_axis_name="c", subcore_axis_name="s")

    @functools.partial(
        pl.kernel,
        mesh=mesh,
        out_type=jax.ShapeDtypeStruct((B, V), jnp.float32),
        scratch_types=[
            pltpu.VMEM((8, MC), jnp.float32),
            pltpu.VMEM((8, MC), jnp.float32),
            pltpu.VMEM((M,), jnp.float32),
            pltpu.VMEM((V,), jnp.float32),
            pltpu.VMEM((V,), jnp.float32),
            pltpu.SemaphoreType.DMA,
            pltpu.SemaphoreType.DMA,
        ],
    )
    def vm_kernel(w_hbm, v_hbm, memt_hbm, out_hbm,
                  buf0, buf1, w_v, v_v, o_v, sem0, sem1):
        b = lax.axis_index("s") * NC + lax.axis_index("c")

        pltpu.sync_copy(w_hbm.at[b], w_v)
        pltpu.sync_copy(v_hbm.at[b], v_v)

        def wsq_body(i, acc):
            x = w_v[pl.ds(i * L, L)]
            return acc + x * x

        wsqv = lax.fori_loop(0, M // L, wsq_body, jnp.zeros((L,), jnp.float32))
        wsq = wsqv[0]
        for i in range(1, L):
            wsq = wsq + wsqv[i]

        _dnums = lax.GatherDimensionNumbers(
            offset_dims=(), collapsed_slice_dims=(0,), start_index_map=(0,))

        lanes = lax.iota(jnp.int32, L)

        def perm(x, idx):
            return lax.gather(x, idx.reshape(L, 1), _dnums, slice_sizes=(1,),
                              mode=lax.GatherScatterMode.PROMISE_IN_BOUNDS)

        def lane_sum(x):
            for step in (8, 4, 2, 1):
                x = x + perm(x, lanes ^ step)
            return x

        def slab(buf, m0, accs):
            def body(m16, accs):
                new = []
                for u in range(2):
                    wv = w_v[pl.ds(m0 + (m16 * 2 + u) * L, L)]
                    for r in range(8):
                        a = accs[r] if u == 0 else new[r]
                        t = a + wv * buf[r, pl.ds((m16 * 2 + u) * L, L)]
                        if u == 0:
                            new.append(t)
                        else:
                            new[r] = t
                return tuple(new)

            return lax.fori_loop(0, MC // L // 2, body, accs)

        zeros8 = tuple(jnp.zeros((L,), jnp.float32) for _ in range(8))
        bufs = (buf0, buf1)
        sems = (sem0, sem1)

        def src(c):
            tv, mchunk = divmod(c, M // MC)
            return memt_hbm.at[b, pl.ds(8 * tv, 8), pl.ds(mchunk * MC, MC)]

        pltpu.async_copy(src(0), bufs[0], sems[0])

        accs = zeros8
        outs = []
        for c in range(NCH):
            if c + 1 < NCH:
                pltpu.async_copy(src(c + 1), bufs[(c + 1) % 2], sems[(c + 1) % 2])
            pltpu.make_async_copy(src(c), bufs[c % 2], sems[c % 2]).wait()
            accs = slab(bufs[c % 2], (c % (M // MC)) * MC, accs)
            if c % (M // MC) == M // MC - 1:
                tv = c // (M // MC)
                group = jnp.zeros((L,), jnp.float32)
                for r in range(8):
                    lane = 8 * (tv % 2) + r
                    onehot = jnp.where(lanes == lane, 1.0, 0.0)
                    group = group + lane_sum(accs[r]) * onehot
                outs.append(group)
                accs = zeros8

        for g in range(4):
            res = outs[2 * g] + outs[2 * g + 1]
            o_v[pl.ds(g * L, L)] = res + wsq * v_v[pl.ds(g * L, L)]
        pltpu.sync_copy(o_v, out_hbm.at[b])

    return vm_kernel


_vm_kernel = _make_kernel()


def kernel(w, v, memory):
    mem_t = jnp.transpose(memory, (0, 2, 1))
    return _vm_kernel(w, v, mem_t)


# speedup vs baseline: 6.1524x; 1.3471x over previous
import functools

import jax
import jax.numpy as jnp
from jax import lax
from jax.experimental import pallas as pl
from jax.experimental.pallas import tpu as pltpu
from jax.experimental.pallas import tpu_sc as plsc

B, M, V = 32, 16384, 64
NC, NS, L = 2, 16, 16
SC_B = 8
TC_B = B - SC_B
WPB = (NC * NS) // SC_B
TVW = (V // 8) // WPB
MC = 4096
MCH = M // MC
NCH = TVW * MCH
MB = 2048
TB = 8


def _make_sc_kernel():
    mesh = plsc.VectorSubcoreMesh(core_axis_name="c", subcore_axis_name="s")

    @functools.partial(
        pl.kernel,
        mesh=mesh,
        out_type=jax.ShapeDtypeStruct((SC_B, V), jnp.float32),
        scratch_types=[
            pltpu.VMEM((8, MC), jnp.float32),
            pltpu.VMEM((8, MC), jnp.float32),
            pltpu.VMEM((M,), jnp.float32),
            pltpu.VMEM((L,), jnp.float32),
            pltpu.VMEM((L,), jnp.float32),
            pltpu.SemaphoreType.DMA,
            pltpu.SemaphoreType.DMA,
        ],
    )
    def sc_kernel(w_hbm, v_hbm, memt_hbm, out_hbm,
                  buf0, buf1, w_v, v_v, o_v, sem0, sem1):
        wid = lax.axis_index("s") * NC + lax.axis_index("c")
        b = wid // WPB
        g = wid % WPB
        bfull = TC_B + b

        pltpu.sync_copy(w_hbm.at[bfull], w_v)
        pltpu.sync_copy(v_hbm.at[bfull, pl.ds(g * L, L)], v_v)

        def wsq_body(i, acc):
            x = w_v[pl.ds(i * L, L)]
            return acc + x * x

        wsqv = lax.fori_loop(0, M // L, wsq_body, jnp.zeros((L,), jnp.float32))
        wsq = wsqv[0]
        for i in range(1, L):
            wsq = wsq + wsqv[i]

        _dnums = lax.GatherDimensionNumbers(
            offset_dims=(), collapsed_slice_dims=(0,), start_index_map=(0,))
        lanes = lax.iota(jnp.int32, L)

        def perm(x, idx):
            return lax.gather(x, idx.reshape(L, 1), _dnums, slice_sizes=(1,),
                              mode=lax.GatherScatterMode.PROMISE_IN_BOUNDS)

        def lane_sum(x):
            for step in (8, 4, 2, 1):
                x = x + perm(x, lanes ^ step)
            return x

        def slab(buf, m0, accs):
            def body(m16, accs):
                new = []
                for u in range(2):
                    wv = w_v[pl.ds(m0 + (m16 * 2 + u) * L, L)]
                    for r in range(8):
                        a = accs[r] if u == 0 else new[r]
                        t = a + wv * buf[r, pl.ds((m16 * 2 + u) * L, L)]
                        if u == 0:
                            new.append(t)
                        else:
                            new[r] = t
                return tuple(new)

            return lax.fori_loop(0, MC // L // 2, body, accs)

        zeros8 = tuple(jnp.zeros((L,), jnp.float32) for _ in range(8))
        bufs = (buf0, buf1)
        sems = (sem0, sem1)

        def src(c):
            tv = c // MCH
            mchunk = c % MCH
            return memt_hbm.at[bfull,
                               pl.ds((g * TVW + tv) * 8, 8),
                               pl.ds(mchunk * MC, MC)]

        pltpu.async_copy(src(0), bufs[0], sems[0])

        accs = zeros8
        outs = []
        for c in range(NCH):
            if c + 1 < NCH:
                pltpu.async_copy(src(c + 1), bufs[(c + 1) % 2], sems[(c + 1) % 2])
            pltpu.make_async_copy(src(c), bufs[c % 2], sems[c % 2]).wait()
            accs = slab(bufs[c % 2], (c % MCH) * MC, accs)
            if c % MCH == MCH - 1:
                tv = c // MCH
                group = jnp.zeros((L,), jnp.float32)
                for r in range(8):
                    lane = 8 * (tv % 2) + r
                    onehot = jnp.where(lanes == lane, 1.0, 0.0)
                    group = group + lane_sum(accs[r]) * onehot
                outs.append(group)
                accs = zeros8

        res = outs[0] + outs[1]
        o_v[...] = res + wsq * v_v[...]
        pltpu.sync_copy(o_v, out_hbm.at[b, pl.ds(g * L, L)])

    return sc_kernel


_sc_kernel = _make_sc_kernel()


def _tc_body(w_ref, v_ref, mem_ref, o_ref):
    mi = pl.program_id(1)

    @pl.when(mi == 0)
    def _():
        o_ref[...] = jnp.zeros_like(o_ref)

    wv = w_ref[...]
    part = lax.dot_general(mem_ref[...], wv, (((2,), (1,)), ((0,), (0,))),
                           preferred_element_type=jnp.float32)
    wsq_part = jnp.sum(wv * wv, axis=1, keepdims=True)
    o_ref[...] += part + wsq_part * v_ref[...]


_tc_call = pl.pallas_call(
    _tc_body,
    grid=(TC_B // TB, M // MB),
    in_specs=[
        pl.BlockSpec((TB, MB), lambda b, mi: (b, mi)),
        pl.BlockSpec((TB, V), lambda b, mi: (b, 0)),
        pl.BlockSpec((TB, V, MB), lambda b, mi: (b, 0, mi)),
    ],
    out_specs=pl.BlockSpec((TB, V), lambda b, mi: (b, 0)),
    out_shape=jax.ShapeDtypeStruct((TC_B, V), jnp.float32),
)


def kernel(w, v, memory):
    mem_t = jnp.transpose(memory, (0, 2, 1))
    out_sc = _sc_kernel(w, v, mem_t)
    out_tc = _tc_call(w, v, mem_t)
    return jnp.concatenate([out_tc, out_sc], axis=0)


# speedup vs baseline: 6.4411x; 1.0469x over previous
import functools

import jax
import jax.numpy as jnp
from jax import lax
from jax.experimental import pallas as pl
from jax.experimental.pallas import tpu as pltpu
from jax.experimental.pallas import tpu_sc as plsc

B, M, V = 32, 16384, 64
NC, NS, L = 2, 16, 16
SC_B = 8
TC_B = B - SC_B
WPB = (NC * NS) // SC_B
TVW = (V // 8) // WPB
MC = 4096
MCH = M // MC
NCH = TVW * MCH
MB = 8192
TB = 8


def _make_sc_kernel():
    mesh = plsc.VectorSubcoreMesh(core_axis_name="c", subcore_axis_name="s")

    @functools.partial(
        pl.kernel,
        mesh=mesh,
        out_type=jax.ShapeDtypeStruct((SC_B, V), jnp.float32),
        scratch_types=[
            pltpu.VMEM((8, MC), jnp.float32),
            pltpu.VMEM((8, MC), jnp.float32),
            pltpu.VMEM((M,), jnp.float32),
            pltpu.VMEM((L,), jnp.float32),
            pltpu.VMEM((L,), jnp.float32),
            pltpu.SemaphoreType.DMA,
            pltpu.SemaphoreType.DMA,
        ],
    )
    def sc_kernel(w_hbm, v_hbm, memt_hbm, out_hbm,
                  buf0, buf1, w_v, v_v, o_v, sem0, sem1):
        wid = lax.axis_index("s") * NC + lax.axis_index("c")
        b = wid // WPB
        g = wid % WPB
        bfull = TC_B + b

        pltpu.sync_copy(w_hbm.at[bfull], w_v)
        pltpu.sync_copy(v_hbm.at[bfull, pl.ds(g * L, L)], v_v)

        def wsq_body(i, acc):
            x = w_v[pl.ds(i * L, L)]
            return acc + x * x

        wsqv = lax.fori_loop(0, M // L, wsq_body, jnp.zeros((L,), jnp.float32))
        wsq = wsqv[0]
        for i in range(1, L):
            wsq = wsq + wsqv[i]

        _dnums = lax.GatherDimensionNumbers(
            offset_dims=(), collapsed_slice_dims=(0,), start_index_map=(0,))
        lanes = lax.iota(jnp.int32, L)

        def perm(x, idx):
            return lax.gather(x, idx.reshape(L, 1), _dnums, slice_sizes=(1,),
                              mode=lax.GatherScatterMode.PROMISE_IN_BOUNDS)

        def lane_sum(x):
            for step in (8, 4, 2, 1):
                x = x + perm(x, lanes ^ step)
            return x

        def slab(buf, m0, accs):
            def body(m16, accs):
                new = []
                for u in range(2):
                    wv = w_v[pl.ds(m0 + (m16 * 2 + u) * L, L)]
                    for r in range(8):
                        a = accs[r] if u == 0 else new[r]
                        t = a + wv * buf[r, pl.ds((m16 * 2 + u) * L, L)]
                        if u == 0:
                            new.append(t)
                        else:
                            new[r] = t
                return tuple(new)

            return lax.fori_loop(0, MC // L // 2, body, accs)

        zeros8 = tuple(jnp.zeros((L,), jnp.float32) for _ in range(8))
        bufs = (buf0, buf1)
        sems = (sem0, sem1)

        def src(c):
            tv = c // MCH
            mchunk = c % MCH
            return memt_hbm.at[bfull,
                               pl.ds((g * TVW + tv) * 8, 8),
                               pl.ds(mchunk * MC, MC)]

        pltpu.async_copy(src(0), bufs[0], sems[0])

        accs = zeros8
        outs = []
        for c in range(NCH):
            if c + 1 < NCH:
                pltpu.async_copy(src(c + 1), bufs[(c + 1) % 2], sems[(c + 1) % 2])
            pltpu.make_async_copy(src(c), bufs[c % 2], sems[c % 2]).wait()
            accs = slab(bufs[c % 2], (c % MCH) * MC, accs)
            if c % MCH == MCH - 1:
                tv = c // MCH
                group = jnp.zeros((L,), jnp.float32)
                for r in range(8):
                    lane = 8 * (tv % 2) + r
                    onehot = jnp.where(lanes == lane, 1.0, 0.0)
                    group = group + lane_sum(accs[r]) * onehot
                outs.append(group)
                accs = zeros8

        res = outs[0] + outs[1]
        o_v[...] = res + wsq * v_v[...]
        pltpu.sync_copy(o_v, out_hbm.at[b, pl.ds(g * L, L)])

    return sc_kernel


_sc_kernel = _make_sc_kernel()


def _tc_body(w_ref, v_ref, mem_ref, o_ref):
    mi = pl.program_id(1)

    @pl.when(mi == 0)
    def _():
        o_ref[...] = jnp.zeros_like(o_ref)

    wv = w_ref[...]
    part = lax.dot_general(mem_ref[...], wv, (((2,), (1,)), ((0,), (0,))),
                           preferred_element_type=jnp.float32)
    wsq_part = jnp.sum(wv * wv, axis=1, keepdims=True)
    o_ref[...] += part + wsq_part * v_ref[...]


_tc_call = pl.pallas_call(
    _tc_body,
    grid=(TC_B // TB, M // MB),
    in_specs=[
        pl.BlockSpec((TB, MB), lambda b, mi: (b, mi)),
        pl.BlockSpec((TB, V), lambda b, mi: (b, 0)),
        pl.BlockSpec((TB, V, MB), lambda b, mi: (b, 0, mi)),
    ],
    out_specs=pl.BlockSpec((TB, V), lambda b, mi: (b, 0)),
    out_shape=jax.ShapeDtypeStruct((TC_B, V), jnp.float32),
)


def kernel(w, v, memory):
    mem_t = jnp.transpose(memory, (0, 2, 1))
    out_sc = _sc_kernel(w, v, mem_t)
    out_tc = _tc_call(w, v, mem_t)
    return jnp.concatenate([out_tc, out_sc], axis=0)
